# baseline (speedup 1.0000x reference)
import jax
import jax.numpy as jnp
from jax import lax
from jax.experimental import pallas as pl
from jax.experimental.pallas import tpu as pltpu
from jax.experimental.pallas import tpu_sc as plsc

N = 10000
NP = 10240
E = 320000
D = 128

NC = 2
NS = 16
K = 80
NB = 4
EP = 327680
RPT = EP // K // NS
NPT = NP // NS

_mesh = plsc.VectorSubcoreMesh(core_axis_name="c", subcore_axis_name="s")


def _deg_body(sd, ones_h, zrows, deg, acc8, ib, onesb, ssem, isem):
  cid = lax.axis_index("c")
  sid = lax.axis_index("s")
  pltpu.sync_copy(ones_h, onesb)
  pltpu.sync_copy(zrows.at[pl.ds(sid * NPT, NPT)],
                  acc8.at[pl.ds(sid * NPT, NPT)])
  sdt = sd.at[cid].at[sid]
  plsc.subcore_barrier()

  def idx_start(c, s, b):
    pltpu.async_copy(sdt.at[c], ib.at[s].at[b], isem.at[s].at[b])

  def idx_wait(c, s, b):
    pltpu.make_async_copy(sdt.at[c], ib.at[s].at[b], isem.at[s].at[b]).wait()

  def sct_start(s, b):
    pltpu.async_copy(onesb, acc8.at[ib.at[s].at[b].at[1]], ssem.at[b],
                     add=True)

  def sct_wait(s, b):
    pltpu.make_async_copy(onesb, acc8.at[ib.at[s].at[b].at[1]],
                          ssem.at[b]).wait()

  for b in range(NB):
    idx_start(b, 0, b)
  for b in range(NB):
    idx_start(NB + b, 1, b)

  def stage(c0, s):
    for b in range(NB):
      idx_wait(c0 + b, s, b)
      sct_start(s, b)
    for b in range(NB):
      sct_wait(s, b)
      nxt = c0 + 2 * NB + b

      @pl.when(nxt < RPT)
      def _(nxt=nxt, s=s, b=b):
        idx_start(nxt, s, b)

  def body(g, c):
    c0 = 2 * NB * g
    stage(c0, 0)
    stage(c0 + NB, 1)
    return c

  lax.fori_loop(0, RPT // (2 * NB), body, 0)
  plsc.subcore_barrier()
  pltpu.sync_copy(acc8.at[pl.ds(sid * NPT, NPT)],
                  deg.at[cid].at[pl.ds(sid * NPT, NPT)])


_deg_call = pl.kernel(
    _deg_body,
    out_type=jax.ShapeDtypeStruct((NC, NP, D), jnp.float32),
    mesh=_mesh,
    scratch_types=[
        pltpu.VMEM_SHARED((NP, D), jnp.float32),
        pltpu.VMEM((2, NB, 2, K), jnp.int32),
        pltpu.VMEM((K, D), jnp.float32),
        pltpu.SemaphoreType.DMA((NB,)),
        pltpu.SemaphoreType.DMA((2, NB)),
    ],
)


def _prop_body(yp, sd, zrows, s_out, acc, ib, rows, gsem, ssem, isem):
  cid = lax.axis_index("c")
  sid = lax.axis_index("s")
  pltpu.sync_copy(zrows.at[pl.ds(sid * NPT, NPT)],
                  acc.at[pl.ds(sid * NPT, NPT)])
  sdt = sd.at[cid].at[sid]
  plsc.subcore_barrier()

  U = 4 * NB

  def S(du):
    return ((du % U) // NB) % 4

  def B(du):
    return (du % U) % NB

  def idx_start(c, du):
    pltpu.async_copy(sdt.at[c], ib.at[S(du)].at[B(du)],
                     isem.at[S(du)].at[B(du)])

  def idx_wait(c, du):
    pltpu.make_async_copy(sdt.at[c], ib.at[S(du)].at[B(du)],
                          isem.at[S(du)].at[B(du)]).wait()

  def gat_start(c, du):
    pltpu.async_copy(yp.at[ib.at[S(du)].at[B(du)].at[0]], rows.at[B(du)],
                     gsem.at[B(du)])

  def gat_wait(c, du):
    pltpu.make_async_copy(yp.at[ib.at[S(du)].at[B(du)].at[0]],
                          rows.at[B(du)], gsem.at[B(du)]).wait()

  def sct_start(c, du):
    pltpu.async_copy(rows.at[B(du)], acc.at[ib.at[S(du)].at[B(du)].at[1]],
                     ssem.at[B(du)], add=True)

  def sct_wait(c, du):
    pltpu.make_async_copy(rows.at[B(du)],
                          acc.at[ib.at[S(du)].at[B(du)].at[1]],
                          ssem.at[B(du)]).wait()

  def chunk_ops(cbase, u, lo_ok, hi_ok):
    c = cbase + u
    if lo_ok:
      sct_wait(c - NB, u - NB)
    idx_wait(c, u)
    gat_start(c, u)
    if lo_ok or u >= NB - 1:
      gat_wait(c - (NB - 1), u - (NB - 1))
      sct_start(c - (NB - 1), u - (NB - 1))
    if hi_ok:
      idx_start(c + 2 * NB, u + 2 * NB)

  for u in range(2 * NB):
    idx_start(u, u)
  for u in range(U):
    chunk_ops(0, u, u >= NB, u + 2 * NB < RPT)

  def body(g, c):
    cbase = U * g
    for u in range(U):
      chunk_ops(cbase, u, True, True)
    return c

  lax.fori_loop(1, RPT // U - 1, body, 0)
  for u in range(U):
    chunk_ops(RPT - U, u, True, u + 2 * NB < U)
  for u in range(U - (NB - 1), U):
    gat_wait(RPT - U + u, u)
    sct_start(RPT - U + u, u)
  for u in range(U - NB, U):
    sct_wait(RPT - U + u, u)

  plsc.subcore_barrier()
  pltpu.sync_copy(acc.at[pl.ds(sid * NPT, NPT)],
                  s_out.at[cid].at[pl.ds(sid * NPT, NPT)])


_prop_call = pl.kernel(
    _prop_body,
    out_type=jax.ShapeDtypeStruct((NC, NP, D), jnp.float32),
    mesh=_mesh,
    scratch_types=[
        pltpu.VMEM_SHARED((NP, D), jnp.float32),
        pltpu.VMEM((4, NB, 2, K), jnp.int32),
        pltpu.VMEM((NB, K, D), jnp.float32),
        pltpu.SemaphoreType.DMA((NB,)),
        pltpu.SemaphoreType.DMA((NB,)),
        pltpu.SemaphoreType.DMA((4, NB)),
    ],
)


BN = 2000
GB = N // BN


def _mm_body(x_ref, w_ref, o_ref):
  o_ref[...] = jnp.dot(x_ref[...], w_ref[...],
                       preferred_element_type=jnp.float32)


def _mm(x, w):
  return pl.pallas_call(
      _mm_body,
      grid=(2 * GB,),
      in_specs=[
          pl.BlockSpec((BN, D), lambda i: (i, 0)),
          pl.BlockSpec((D, D), lambda i: (0, 0)),
      ],
      out_specs=pl.BlockSpec((BN, D), lambda i: (i, 0)),
      out_shape=jax.ShapeDtypeStruct((2 * N, D), jnp.float32),
  )(x, w)


def _scale_body(u_ref, deg_ref, o_ref):
  dinv = lax.rsqrt(deg_ref[0, :, 0:1] + 1.0)
  o_ref[...] = u_ref[...] * dinv


def _scale(u, deg):
  return pl.pallas_call(
      _scale_body,
      grid=(2 * GB,),
      in_specs=[
          pl.BlockSpec((BN, D), lambda i: (i, 0)),
          pl.BlockSpec((1, BN, D), lambda i: (i // GB, i % GB, 0)),
      ],
      out_specs=pl.BlockSpec((BN, D), lambda i: (i, 0)),
      out_shape=jax.ShapeDtypeStruct((2 * N, D), jnp.float32),
  )(u, deg)


def _mid_body(s_ref, y_ref, deg_ref, b_ref, w_ref, o_ref):
  dinv = lax.rsqrt(deg_ref[0, :, 0:1] + 1.0)
  h = jnp.maximum((s_ref[0] + y_ref[...]) * dinv + b_ref[...], 0.0)
  o_ref[...] = jnp.dot(h, w_ref[...],
                       preferred_element_type=jnp.float32) * dinv


def _mid(s, y, deg, b, w):
  return pl.pallas_call(
      _mid_body,
      grid=(2 * GB,),
      in_specs=[
          pl.BlockSpec((1, BN, D), lambda i: (i // GB, i % GB, 0)),
          pl.BlockSpec((BN, D), lambda i: (i, 0)),
          pl.BlockSpec((1, BN, D), lambda i: (i // GB, i % GB, 0)),
          pl.BlockSpec((1, D), lambda i: (0, 0)),
          pl.BlockSpec((D, D), lambda i: (0, 0)),
      ],
      out_specs=pl.BlockSpec((BN, D), lambda i: (i, 0)),
      out_shape=jax.ShapeDtypeStruct((2 * N, D), jnp.float32),
  )(s, y, deg, b, w)


def _final_body(s0_ref, s1_ref, y0_ref, y1_ref, deg0_ref, deg1_ref, b_ref,
                o0_ref, o1_ref):
  b = b_ref[...]
  dinv0 = lax.rsqrt(deg0_ref[0, :, 0:1] + 1.0)
  o0_ref[...] = (s0_ref[0] + y0_ref[...]) * dinv0 + b
  dinv1 = lax.rsqrt(deg1_ref[0, :, 0:1] + 1.0)
  o1_ref[...] = (s1_ref[0] + y1_ref[...]) * dinv1 + b


def _final(s, y, deg, b):
  return pl.pallas_call(
      _final_body,
      grid=(GB,),
      in_specs=[
          pl.BlockSpec((1, BN, D), lambda i: (0, i, 0)),
          pl.BlockSpec((1, BN, D), lambda i: (1, i, 0)),
          pl.BlockSpec((BN, D), lambda i: (i, 0)),
          pl.BlockSpec((BN, D), lambda i: (i + GB, 0)),
          pl.BlockSpec((1, BN, D), lambda i: (0, i, 0)),
          pl.BlockSpec((1, BN, D), lambda i: (1, i, 0)),
          pl.BlockSpec((1, D), lambda i: (0, 0)),
      ],
      out_specs=[
          pl.BlockSpec((BN, D), lambda i: (i, 0)),
          pl.BlockSpec((BN, D), lambda i: (i, 0)),
      ],
      out_shape=[
          jax.ShapeDtypeStruct((N, D), jnp.float32),
          jax.ShapeDtypeStruct((N, D), jnp.float32),
      ],
  )(s, s, y, y, deg, deg, b)


def kernel(x1, edge_index1, x2, edge_index2, W1, b1, W2, b2):
  idt = edge_index1.dtype
  pad1 = jnp.zeros((EP - E,), idt)
  pad_dst = jnp.full((EP - E,), N, idt)
  src1 = jnp.concatenate([edge_index1[0], pad1]).reshape(NS, RPT, K)
  dst1 = jnp.concatenate([edge_index1[1], pad_dst]).reshape(NS, RPT, K)
  src2 = (jnp.concatenate([edge_index2[0], pad1]) + N).reshape(NS, RPT, K)
  dst2 = jnp.concatenate([edge_index2[1], pad_dst]).reshape(NS, RPT, K)
  sd1 = jnp.stack([src1, dst1], axis=2)
  sd2 = jnp.stack([src2, dst2], axis=2)
  sd = jnp.stack([sd1, sd2], axis=0)

  xcat = jnp.concatenate([x1, x2], axis=0)
  ones_h = jnp.ones((K, D), jnp.float32)
  zrows = jnp.zeros((NP, D), jnp.float32)
  b1r = b1.reshape(1, D)
  b2r = b2.reshape(1, D)

  u1 = _mm(xcat, W1)
  deg = _deg_call(sd, ones_h, zrows)
  y1 = _scale(u1, deg)
  s1 = _prop_call(y1, sd, zrows)
  y2 = _mid(s1, y1, deg, b1r, W2)
  s2 = _prop_call(y2, sd, zrows)
  z1, z2 = _final(s2, y2, deg, b2r)
  return (z1, z2)


# speedup vs baseline: 1.0296x; 1.0296x over previous
import jax
import jax.numpy as jnp
from jax import lax
from jax.experimental import pallas as pl
from jax.experimental.pallas import tpu as pltpu
from jax.experimental.pallas import tpu_sc as plsc

N = 10000
NP = 10240
E = 320000
D = 128

NC = 2
NS = 16
K = 80
NB = 4
EP = 327680
RPT = EP // K // NS
NPT = NP // NS

_mesh = plsc.VectorSubcoreMesh(core_axis_name="c", subcore_axis_name="s")


def _deg_body(sd, ones_h, zrows, deg, acc8, ib, onesb, ssem, isem):
  cid = lax.axis_index("c")
  sid = lax.axis_index("s")
  pltpu.sync_copy(ones_h, onesb)
  pltpu.sync_copy(zrows.at[pl.ds(sid * NPT, NPT)],
                  acc8.at[pl.ds(sid * NPT, NPT)])
  sdt = sd.at[cid].at[sid]
  plsc.subcore_barrier()

  def idx_start(c, s, b):
    pltpu.async_copy(sdt.at[c], ib.at[s].at[b], isem.at[s].at[b])

  def idx_wait(c, s, b):
    pltpu.make_async_copy(sdt.at[c], ib.at[s].at[b], isem.at[s].at[b]).wait()

  def sct_start(s, b):
    pltpu.async_copy(onesb, acc8.at[ib.at[s].at[b].at[1]], ssem.at[b],
                     add=True)

  def sct_wait(s, b):
    pltpu.make_async_copy(onesb, acc8.at[ib.at[s].at[b].at[1]],
                          ssem.at[b]).wait()

  for b in range(NB):
    idx_start(b, 0, b)
  for b in range(NB):
    idx_start(NB + b, 1, b)

  def stage(c0, s):
    for b in range(NB):
      idx_wait(c0 + b, s, b)
      sct_start(s, b)
    for b in range(NB):
      sct_wait(s, b)
      nxt = c0 + 2 * NB + b

      @pl.when(nxt < RPT)
      def _(nxt=nxt, s=s, b=b):
        idx_start(nxt, s, b)

  def body(g, c):
    c0 = 2 * NB * g
    stage(c0, 0)
    stage(c0 + NB, 1)
    return c

  lax.fori_loop(0, RPT // (2 * NB), body, 0)
  plsc.subcore_barrier()
  pltpu.sync_copy(acc8.at[pl.ds(sid * NPT, NPT)],
                  deg.at[cid].at[pl.ds(sid * NPT, NPT)])


_deg_call = pl.kernel(
    _deg_body,
    out_type=jax.ShapeDtypeStruct((NC, NP, D), jnp.float32),
    mesh=_mesh,
    scratch_types=[
        pltpu.VMEM_SHARED((NP, D), jnp.float32),
        pltpu.VMEM((2, NB, 2, K), jnp.int32),
        pltpu.VMEM((K, D), jnp.float32),
        pltpu.SemaphoreType.DMA((NB,)),
        pltpu.SemaphoreType.DMA((2, NB)),
    ],
)


def _prop_body(yp, sd, zrows, s_out, acc, ib, rows, gsem, ssem, isem):
  cid = lax.axis_index("c")
  sid = lax.axis_index("s")
  pltpu.sync_copy(zrows.at[pl.ds(sid * NPT, NPT)],
                  acc.at[pl.ds(sid * NPT, NPT)])
  sdt = sd.at[cid].at[sid]
  plsc.subcore_barrier()

  U = 4 * NB

  def S(du):
    return ((du % U) // NB) % 4

  def B(du):
    return (du % U) % NB

  def idx_start(c, du):
    pltpu.async_copy(sdt.at[c], ib.at[S(du)].at[B(du)],
                     isem.at[S(du)].at[B(du)])

  def idx_wait(c, du):
    pltpu.make_async_copy(sdt.at[c], ib.at[S(du)].at[B(du)],
                          isem.at[S(du)].at[B(du)]).wait()

  def gat_start(c, du):
    pltpu.async_copy(yp.at[ib.at[S(du)].at[B(du)].at[0]], rows.at[B(du)],
                     gsem.at[B(du)])

  def gat_wait(c, du):
    pltpu.make_async_copy(yp.at[ib.at[S(du)].at[B(du)].at[0]],
                          rows.at[B(du)], gsem.at[B(du)]).wait()

  def sct_start(c, du):
    pltpu.async_copy(rows.at[B(du)], acc.at[ib.at[S(du)].at[B(du)].at[1]],
                     ssem.at[B(du)], add=True)

  def sct_wait(c, du):
    pltpu.make_async_copy(rows.at[B(du)],
                          acc.at[ib.at[S(du)].at[B(du)].at[1]],
                          ssem.at[B(du)]).wait()

  def chunk_ops(cbase, u, lo_ok, hi_ok):
    c = cbase + u
    if lo_ok:
      sct_wait(c - NB, u - NB)
    idx_wait(c, u)
    gat_start(c, u)
    if lo_ok or u >= NB - 1:
      gat_wait(c - (NB - 1), u - (NB - 1))
      sct_start(c - (NB - 1), u - (NB - 1))
    if hi_ok:
      idx_start(c + 2 * NB, u + 2 * NB)

  for u in range(2 * NB):
    idx_start(u, u)
  for u in range(U):
    chunk_ops(0, u, u >= NB, u + 2 * NB < RPT)

  def body(g, c):
    cbase = U * g
    for u in range(U):
      chunk_ops(cbase, u, True, True)
    return c

  lax.fori_loop(1, RPT // U - 1, body, 0)
  for u in range(U):
    chunk_ops(RPT - U, u, True, u + 2 * NB < U)
  for u in range(U - (NB - 1), U):
    gat_wait(RPT - U + u, u)
    sct_start(RPT - U + u, u)
  for u in range(U - NB, U):
    sct_wait(RPT - U + u, u)

  plsc.subcore_barrier()
  pltpu.sync_copy(acc.at[pl.ds(sid * NPT, NPT)],
                  s_out.at[cid].at[pl.ds(sid * NPT, NPT)])


_prop_call = pl.kernel(
    _prop_body,
    out_type=jax.ShapeDtypeStruct((NC, NP, D), jnp.float32),
    mesh=_mesh,
    scratch_types=[
        pltpu.VMEM_SHARED((NP, D), jnp.float32),
        pltpu.VMEM((4, NB, 2, K), jnp.int32),
        pltpu.VMEM((NB, K, D), jnp.float32),
        pltpu.SemaphoreType.DMA((NB,)),
        pltpu.SemaphoreType.DMA((NB,)),
        pltpu.SemaphoreType.DMA((4, NB)),
    ],
)


BN = 2000
GB = N // BN


def _mm_scale_body(x_ref, w_ref, deg_ref, o_ref):
  dinv = lax.rsqrt(deg_ref[0, :, 0:1] + 1.0)
  o_ref[...] = jnp.dot(x_ref[...], w_ref[...],
                       preferred_element_type=jnp.float32) * dinv


def _mm_scale(x, w, deg):
  return pl.pallas_call(
      _mm_scale_body,
      grid=(2 * GB,),
      in_specs=[
          pl.BlockSpec((BN, D), lambda i: (i, 0)),
          pl.BlockSpec((D, D), lambda i: (0, 0)),
          pl.BlockSpec((1, BN, D), lambda i: (i // GB, i % GB, 0)),
      ],
      out_specs=pl.BlockSpec((BN, D), lambda i: (i, 0)),
      out_shape=jax.ShapeDtypeStruct((2 * N, D), jnp.float32),
  )(x, w, deg)


def _mid_body(s_ref, y_ref, deg_ref, b_ref, w_ref, o_ref):
  dinv = lax.rsqrt(deg_ref[0, :, 0:1] + 1.0)
  h = jnp.maximum((s_ref[0] + y_ref[...]) * dinv + b_ref[...], 0.0)
  o_ref[...] = jnp.dot(h, w_ref[...],
                       preferred_element_type=jnp.float32) * dinv


def _mid(s, y, deg, b, w):
  return pl.pallas_call(
      _mid_body,
      grid=(2 * GB,),
      in_specs=[
          pl.BlockSpec((1, BN, D), lambda i: (i // GB, i % GB, 0)),
          pl.BlockSpec((BN, D), lambda i: (i, 0)),
          pl.BlockSpec((1, BN, D), lambda i: (i // GB, i % GB, 0)),
          pl.BlockSpec((1, D), lambda i: (0, 0)),
          pl.BlockSpec((D, D), lambda i: (0, 0)),
      ],
      out_specs=pl.BlockSpec((BN, D), lambda i: (i, 0)),
      out_shape=jax.ShapeDtypeStruct((2 * N, D), jnp.float32),
  )(s, y, deg, b, w)


def _final_body(s0_ref, s1_ref, y0_ref, y1_ref, deg0_ref, deg1_ref, b_ref,
                o0_ref, o1_ref):
  b = b_ref[...]
  dinv0 = lax.rsqrt(deg0_ref[0, :, 0:1] + 1.0)
  o0_ref[...] = (s0_ref[0] + y0_ref[...]) * dinv0 + b
  dinv1 = lax.rsqrt(deg1_ref[0, :, 0:1] + 1.0)
  o1_ref[...] = (s1_ref[0] + y1_ref[...]) * dinv1 + b


def _final(s, y, deg, b):
  return pl.pallas_call(
      _final_body,
      grid=(GB,),
      in_specs=[
          pl.BlockSpec((1, BN, D), lambda i: (0, i, 0)),
          pl.BlockSpec((1, BN, D), lambda i: (1, i, 0)),
          pl.BlockSpec((BN, D), lambda i: (i, 0)),
          pl.BlockSpec((BN, D), lambda i: (i + GB, 0)),
          pl.BlockSpec((1, BN, D), lambda i: (0, i, 0)),
          pl.BlockSpec((1, BN, D), lambda i: (1, i, 0)),
          pl.BlockSpec((1, D), lambda i: (0, 0)),
      ],
      out_specs=[
          pl.BlockSpec((BN, D), lambda i: (i, 0)),
          pl.BlockSpec((BN, D), lambda i: (i, 0)),
      ],
      out_shape=[
          jax.ShapeDtypeStruct((N, D), jnp.float32),
          jax.ShapeDtypeStruct((N, D), jnp.float32),
      ],
  )(s, s, y, y, deg, deg, b)


def kernel(x1, edge_index1, x2, edge_index2, W1, b1, W2, b2):
  idt = edge_index1.dtype
  pad1 = jnp.zeros((EP - E,), idt)
  pad_dst = jnp.full((EP - E,), N, idt)
  src1 = jnp.concatenate([edge_index1[0], pad1]).reshape(NS, RPT, K)
  dst1 = jnp.concatenate([edge_index1[1], pad_dst]).reshape(NS, RPT, K)
  src2 = (jnp.concatenate([edge_index2[0], pad1]) + N).reshape(NS, RPT, K)
  dst2 = jnp.concatenate([edge_index2[1], pad_dst]).reshape(NS, RPT, K)
  sd1 = jnp.stack([src1, dst1], axis=2)
  sd2 = jnp.stack([src2, dst2], axis=2)
  sd = jnp.stack([sd1, sd2], axis=0)

  xcat = jnp.concatenate([x1, x2], axis=0)
  ones_h = jnp.ones((K, D), jnp.float32)
  zrows = jnp.zeros((NP, D), jnp.float32)
  b1r = b1.reshape(1, D)
  b2r = b2.reshape(1, D)

  deg = _deg_call(sd, ones_h, zrows)
  y1 = _mm_scale(xcat, W1, deg)
  s1 = _prop_call(y1, sd, zrows)
  y2 = _mid(s1, y1, deg, b1r, W2)
  s2 = _prop_call(y2, sd, zrows)
  z1, z2 = _final(s2, y2, deg, b2r)
  return (z1, z2)
